# baseline (speedup 1.0000x reference)
import functools

import jax
import jax.numpy as jnp
from jax import lax
from jax.experimental import pallas as pl
from jax.experimental.pallas import tpu as pltpu
from jax.experimental.pallas import tpu_sc as plsc

_N = 100000
_C = 128
_NBINS = 20

_BR = 25000
_NB = _N // _BR

_NC = 2
_NS = 16
_NW = _NC * _NS
_CHUNK = 3136
_LAST_BASE = _N - _CHUNK
_VECS = _CHUNK // 16
_UNROLL = 14


def _make_scale_kernel():
    mesh = plsc.VectorSubcoreMesh(core_axis_name="c", subcore_axis_name="s")

    @functools.partial(
        pl.kernel,
        mesh=mesh,
        out_type=jax.ShapeDtypeStruct((_N,), jnp.float32),
        scratch_types=[
            pltpu.VMEM((_CHUNK,), jnp.int32),
            pltpu.VMEM((_CHUNK,), jnp.float32),
            pltpu.VMEM((32,), jnp.float32),
            pltpu.SemaphoreType.DMA,
            pltpu.SemaphoreType.DMA,
        ],
    )
    def scale_k(bins_hbm, temp_hbm, out_hbm, idx_v, t_v, temp_v, sem_t, sem_b):
        wid = lax.axis_index("s") * _NC + lax.axis_index("c")
        base = jnp.minimum(wid * _CHUNK, _LAST_BASE)
        tcopy = pltpu.async_copy(temp_hbm, temp_v.at[pl.ds(0, _NBINS)], sem_t)
        bcopy = pltpu.async_copy(bins_hbm.at[pl.ds(base, _CHUNK)], idx_v, sem_b)
        tcopy.wait()
        t0 = temp_v[pl.ds(0, 16)]
        t1 = temp_v[pl.ds(16, 16)]
        inv0 = 1.0 / (jnp.maximum(t0, 0.0) + 1e-8)
        inv1 = 1.0 / (jnp.maximum(t1, 0.0) + 1e-8)
        bcopy.wait()

        def body(i, carry):
            for u in range(_UNROLL):
                off = (i * _UNROLL + u) * 16
                idx = idx_v[pl.ds(off, 16)]
                idxm = idx & 15
                g0 = inv0.at[idxm].get(mode="promise_in_bounds")
                g1 = inv1.at[idxm].get(mode="promise_in_bounds")
                t_v[pl.ds(off, 16)] = jnp.where(idx < 16, g0, g1)
            return carry

        lax.fori_loop(0, _VECS // _UNROLL, body, 0)
        pltpu.sync_copy(t_v, out_hbm.at[pl.ds(base, _CHUNK)])

    return scale_k


_scale_kernel = _make_scale_kernel()


_CR = 4000
_NCH = _N // _CR
_NBUF = 6


def _tc_pipe_body(scale_hbm, logits_hbm, out_hbm,
                  in_buf, s_buf, out_buf, in_sem, s_sem, out_sem):
    def in_copy(g, buf):
        return pltpu.make_async_copy(
            logits_hbm.at[pl.ds(g * _CR, _CR)], in_buf.at[buf], in_sem.at[buf])

    def out_copy(g, buf):
        return pltpu.make_async_copy(
            out_buf.at[buf], out_hbm.at[pl.ds(g * _CR, _CR)], out_sem.at[buf])

    s_all = pltpu.make_async_copy(scale_hbm, s_buf, s_sem)
    s_all.start()
    for b in range(_NBUF - 1):
        in_copy(b, b).start()
    s_all.wait()

    def body(g, carry):
        buf = lax.rem(g, _NBUF)
        nxt = g + _NBUF - 1

        @pl.when(nxt < _NCH)
        def _():
            in_copy(nxt, lax.rem(nxt, _NBUF)).start()

        in_copy(g, buf).wait()

        @pl.when(g >= _NBUF)
        def _():
            out_copy(g - _NBUF, buf).wait()

        s_col = s_buf[g].reshape(_CR, 1)
        out_buf[buf] = in_buf[buf] * s_col
        out_copy(g, buf).start()
        return carry

    lax.fori_loop(0, _NCH, body, 0)
    for k in range(_NCH - _NBUF, _NCH):
        out_copy(k, k % _NBUF).wait()


def kernel(logits, temperature, bin_assignments):
    bins = bin_assignments.astype(jnp.int32)
    scale = _scale_kernel(bins, temperature)
    scale3d = scale.reshape(_NCH, 1, _CR)
    out = pl.pallas_call(
        _tc_pipe_body,
        in_specs=[
            pl.BlockSpec(memory_space=pl.ANY),
            pl.BlockSpec(memory_space=pl.ANY),
        ],
        out_specs=pl.BlockSpec(memory_space=pl.ANY),
        out_shape=jax.ShapeDtypeStruct((_N, _C), jnp.float32),
        scratch_shapes=[
            pltpu.VMEM((_NBUF, _CR, _C), jnp.float32),
            pltpu.VMEM((_NCH, 1, _CR), jnp.float32),
            pltpu.VMEM((_NBUF, _CR, _C), jnp.float32),
            pltpu.SemaphoreType.DMA((_NBUF,)),
            pltpu.SemaphoreType.DMA,
            pltpu.SemaphoreType.DMA((_NBUF,)),
        ],
    )(scale3d, logits)
    return out


# speedup vs baseline: 1.0160x; 1.0160x over previous
import functools

import jax
import jax.numpy as jnp
from jax import lax
from jax.experimental import pallas as pl
from jax.experimental.pallas import tpu as pltpu
from jax.experimental.pallas import tpu_sc as plsc

_N = 100000
_C = 128
_NBINS = 20

_BR = 25000
_NB = _N // _BR

_NC = 2
_NS = 16
_NW = _NC * _NS
_CHUNK = 3136
_LAST_BASE = _N - _CHUNK
_VECS = _CHUNK // 16
_UNROLL = 14


def _make_scale_kernel():
    mesh = plsc.VectorSubcoreMesh(core_axis_name="c", subcore_axis_name="s")

    @functools.partial(
        pl.kernel,
        mesh=mesh,
        out_type=jax.ShapeDtypeStruct((_N,), jnp.float32),
        scratch_types=[
            pltpu.VMEM((_CHUNK,), jnp.int32),
            pltpu.VMEM((_CHUNK,), jnp.float32),
            pltpu.VMEM((32,), jnp.float32),
            pltpu.SemaphoreType.DMA,
            pltpu.SemaphoreType.DMA,
        ],
    )
    def scale_k(bins_hbm, temp_hbm, out_hbm, idx_v, t_v, temp_v, sem_t, sem_b):
        wid = lax.axis_index("s") * _NC + lax.axis_index("c")
        base = jnp.minimum(wid * _CHUNK, _LAST_BASE)
        tcopy = pltpu.async_copy(temp_hbm, temp_v.at[pl.ds(0, _NBINS)], sem_t)
        bcopy = pltpu.async_copy(bins_hbm.at[pl.ds(base, _CHUNK)], idx_v, sem_b)
        tcopy.wait()
        t0 = temp_v[pl.ds(0, 16)]
        t1 = temp_v[pl.ds(16, 16)]
        inv0 = 1.0 / (jnp.maximum(t0, 0.0) + 1e-8)
        inv1 = 1.0 / (jnp.maximum(t1, 0.0) + 1e-8)
        bcopy.wait()

        def body(i, carry):
            for u in range(_UNROLL):
                off = (i * _UNROLL + u) * 16
                idx = idx_v[pl.ds(off, 16)]
                idxm = idx & 15
                g0 = inv0.at[idxm].get(mode="promise_in_bounds")
                g1 = inv1.at[idxm].get(mode="promise_in_bounds")
                t_v[pl.ds(off, 16)] = jnp.where(idx < 16, g0, g1)
            return carry

        lax.fori_loop(0, _VECS // _UNROLL, body, 0)
        pltpu.sync_copy(t_v, out_hbm.at[pl.ds(base, _CHUNK)])

    return scale_k


_scale_kernel = _make_scale_kernel()


_CR = 5000
_NCH = _N // _CR
_NBUF = 8


def _tc_pipe_body(scale_hbm, logits_hbm, out_hbm,
                  in_buf, s_buf, out_buf, in_sem, s_sem, out_sem):
    def in_copy(g, buf):
        return pltpu.make_async_copy(
            logits_hbm.at[pl.ds(g * _CR, _CR)], in_buf.at[buf], in_sem.at[buf])

    def out_copy(g, buf):
        return pltpu.make_async_copy(
            out_buf.at[buf], out_hbm.at[pl.ds(g * _CR, _CR)], out_sem.at[buf])

    s_all = pltpu.make_async_copy(scale_hbm, s_buf, s_sem)
    s_all.start()
    for b in range(_NBUF - 1):
        in_copy(b, b).start()
    s_all.wait()

    def body(g, carry):
        buf = lax.rem(g, _NBUF)
        nxt = g + _NBUF - 1

        @pl.when(nxt < _NCH)
        def _():
            in_copy(nxt, lax.rem(nxt, _NBUF)).start()

        in_copy(g, buf).wait()

        @pl.when(g >= _NBUF)
        def _():
            out_copy(g - _NBUF, buf).wait()

        s_col = s_buf[g].reshape(_CR, 1)
        out_buf[buf] = in_buf[buf] * s_col
        out_copy(g, buf).start()
        return carry

    lax.fori_loop(0, _NCH, body, 0)
    for k in range(_NCH - _NBUF, _NCH):
        out_copy(k, k % _NBUF).wait()


def kernel(logits, temperature, bin_assignments):
    bins = bin_assignments.astype(jnp.int32)
    scale = _scale_kernel(bins, temperature)
    scale3d = scale.reshape(_NCH, 1, _CR)
    out = pl.pallas_call(
        _tc_pipe_body,
        in_specs=[
            pl.BlockSpec(memory_space=pl.ANY),
            pl.BlockSpec(memory_space=pl.ANY),
        ],
        out_specs=pl.BlockSpec(memory_space=pl.ANY),
        out_shape=jax.ShapeDtypeStruct((_N, _C), jnp.float32),
        scratch_shapes=[
            pltpu.VMEM((_NBUF, _CR, _C), jnp.float32),
            pltpu.VMEM((_NCH, 1, _CR), jnp.float32),
            pltpu.VMEM((_NBUF, _CR, _C), jnp.float32),
            pltpu.SemaphoreType.DMA((_NBUF,)),
            pltpu.SemaphoreType.DMA,
            pltpu.SemaphoreType.DMA((_NBUF,)),
        ],
    )(scale3d, logits)
    return out
